# baseline (speedup 1.0000x reference)
import functools

import jax
import jax.numpy as jnp
from jax import lax
from jax.experimental import pallas as pl
from jax.experimental.pallas import tpu as pltpu
from jax.experimental.pallas import tpu_sc as plsc

NC = 2
NS = 16
LANES = 16
DH = 32
RB = 128
SB = 8
NBUF = 4
N_LAYERS = 3
F32 = jnp.float32
BF16 = jnp.bfloat16
I32 = jnp.int32

PERM = tuple(range(0, DH, 2)) + tuple(range(1, DH, 2))
INV_PERM = tuple(sorted(range(DH), key=lambda i: PERM[i]))


@functools.lru_cache(maxsize=None)
def _build(Npad, NBp):
  NSB = NBp // SB
  RPT = Npad // NS
  NRC = RPT // RB

  mesh = plsc.VectorSubcoreMesh(
      core_axis_name="c", subcore_axis_name="s", num_cores=NC, num_subcores=NS
  )

  @functools.partial(
      pl.kernel,
      out_type=(
          jax.ShapeDtypeStruct((NC, Npad, DH), F32),
          jax.ShapeDtypeStruct((NC, Npad, DH // 2), I32),
          jax.ShapeDtypeStruct((NC, Npad, DH // 2), I32),
      ),
      mesh=mesh,
      compiler_params=pltpu.CompilerParams(
          needs_layout_passes=False, use_tc_tiling_on_sc=False
      ),
      scratch_types=dict(
          acc=pltpu.VMEM_SHARED((Npad, DH), F32),
          srcb=pltpu.VMEM((SB, RB), I32),
          dstb=pltpu.VMEM((SB, RB), I32),
          wb=pltpu.VMEM((SB * RB,), F32),
          gb0=pltpu.VMEM((RB, DH // 2), I32),
          gb1=pltpu.VMEM((RB, DH // 2), I32),
          gb2=pltpu.VMEM((RB, DH // 2), I32),
          gb3=pltpu.VMEM((RB, DH // 2), I32),
          sb0=pltpu.VMEM((RB, DH), F32),
          sb1=pltpu.VMEM((RB, DH), F32),
          sb2=pltpu.VMEM((RB, DH), F32),
          sb3=pltpu.VMEM((RB, DH), F32),
          gs0=pltpu.SemaphoreType.DMA,
          gs1=pltpu.SemaphoreType.DMA,
          gs2=pltpu.SemaphoreType.DMA,
          gs3=pltpu.SemaphoreType.DMA,
          ss0=pltpu.SemaphoreType.DMA,
          ss1=pltpu.SemaphoreType.DMA,
          ss2=pltpu.SemaphoreType.DMA,
          ss3=pltpu.SemaphoreType.DMA,
      ),
  )
  def k(x0, x0h, src3, dst3, w3, xsum, xa, xb, acc, srcb, dstb, wb, gb0, gb1,
        gb2, gb3, sb0, sb1, sb2, sb3, gs0, gs1, gs2, gs3, ss0, ss1, ss2, ss3):
    zb = sb0
    accb = sb1
    sumb = sb2
    hbuf = gb0
    c = lax.axis_index("c")
    s = lax.axis_index("s")
    row0_t = s * RPT

    zeros = jnp.zeros((LANES,), F32)

    def zb_body(r, carry):
      zb[r, pl.ds(0, LANES)] = zeros
      zb[r, pl.ds(LANES, LANES)] = zeros
      return carry

    lax.fori_loop(0, RB, zb_body, 0)

    def zacc_body(kk, carry):
      pltpu.sync_copy(zb, acc.at[pl.ds(row0_t + kk * RB, RB), :])
      return carry

    lax.fori_loop(0, NRC, zacc_body, 0)
    plsc.subcore_barrier()

    splat_dnums = lax.GatherDimensionNumbers(
        offset_dims=(), collapsed_slice_dims=(0,), start_index_map=(0,)
    )
    splat_idx = [jnp.full((LANES, 1), j, I32) for j in range(LANES)]

    def splat(vec, j):
      return lax.gather(
          vec,
          splat_idx[j],
          splat_dnums,
          (1,),
          mode=lax.GatherScatterMode.PROMISE_IN_BOUNDS,
      )

    def edge_pass(xprev_h):
      gbufs = [gb0, gb1, gb2, gb3]
      sbufs = [sb0, sb1, sb2, sb3]
      gsems = [gs0, gs1, gs2, gs3]
      ssems = [ss0, ss1, ss2, ss3]

      def sb_body(q, carry):
        base = q * SB
        pltpu.sync_copy(src3.at[s, pl.ds(base, SB), :], srcb)
        pltpu.sync_copy(dst3.at[s, pl.ds(base, SB), :], dstb)
        pltpu.sync_copy(w3.at[s, pl.ds(base * RB, SB * RB)], wb)

        gd = {}
        sd = {}

        def start_gather(b):
          cur = b % NBUF
          gd[b] = pltpu.async_copy(
              xprev_h.at[c].at[srcb.at[b]], gbufs[cur], gsems[cur]
          )

        for i in range(NBUF - 1):
          start_gather(i)

        for b in range(SB):
          cur = b % NBUF
          gd[b].wait()
          if b + NBUF - 1 < SB:
            start_gather(b + NBUF - 1)
          if b >= NBUF:
            sd[b - NBUF].wait()

          def g_body(g, carry2, b=b, gbuf=gbufs[cur], sbuf=sbufs[cur]):
            e0 = g * LANES
            wvec = wb[pl.ds(b * RB + e0, LANES)]
            for j in range(LANES):
              w = splat(wvec, j)
              ab32 = gbuf[e0 + j, pl.ds(0, LANES)]
              ab = plsc.bitcast(ab32, BF16)
              lo, hi = plsc.unpack(ab, format=plsc.PackFormat.INTERLEAVED)
              sbuf[e0 + j, pl.ds(0, LANES)] = lo * w
              sbuf[e0 + j, pl.ds(LANES, LANES)] = hi * w
            return carry2

          lax.fori_loop(0, RB // LANES, g_body, 0)
          sd[b] = pltpu.async_copy(
              sbufs[cur], acc.at[dstb.at[b]], ssems[cur], add=True
          )
        for b in range(SB - NBUF, SB):
          if b >= 0:
            sd[b].wait()
        return carry

      lax.fori_loop(0, NSB, sb_body, 0)

    def writeback(layer, xnew):
      final = layer == N_LAYERS - 1
      scale = jnp.float32(1.0 / (N_LAYERS + 1))
      lax.fori_loop(0, RB, zb_body, 0)

      def k_body(kk, carry):
        row0 = row0_t + kk * RB
        pltpu.sync_copy(acc.at[pl.ds(row0, RB), :], accb)
        pltpu.sync_copy(zb, acc.at[pl.ds(row0, RB), :])
        if layer == 0:
          pltpu.sync_copy(x0.at[c, pl.ds(row0, RB), :], sumb)
        else:
          pltpu.sync_copy(xsum.at[c, pl.ds(row0, RB), :], sumb)

        def r_body(r, carry2):
          a0 = accb[r, pl.ds(0, LANES)]
          a1 = accb[r, pl.ds(LANES, LANES)]
          t0 = sumb[r, pl.ds(0, LANES)] + a0
          t1 = sumb[r, pl.ds(LANES, LANES)] + a1
          if final:
            t0 = t0 * scale
            t1 = t1 * scale
          else:
            pk = plsc.pack(a0, a1, format=plsc.PackFormat.INTERLEAVED)
            hbuf[r, pl.ds(0, LANES)] = plsc.bitcast(pk, I32)
          sumb[r, pl.ds(0, LANES)] = t0
          sumb[r, pl.ds(LANES, LANES)] = t1
          return carry2

        lax.fori_loop(0, RB, r_body, 0)
        if not final:
          pltpu.sync_copy(hbuf, xnew.at[c, pl.ds(row0, RB), :])
        pltpu.sync_copy(sumb, xsum.at[c, pl.ds(row0, RB), :])
        return carry

      lax.fori_loop(0, NRC, k_body, 0)

    xprevs = [x0h, xa, xb]
    xnews = [xa, xb, xa]
    for layer in range(N_LAYERS):
      edge_pass(xprevs[layer])
      plsc.subcore_barrier()
      writeback(layer, xnews[layer])
      plsc.subcore_barrier()

  return k


@jax.jit
def _lightgcn(user_emb, item_emb, edge_index, edge_weight):
  nu = user_emb.shape[0]
  ni = item_emb.shape[0]
  n = nu + ni
  ego = jnp.concatenate([user_emb, item_emb], axis=0)
  npad = -(-n // (NS * RB)) * (NS * RB)
  ego = jnp.pad(ego, ((0, npad - n), (0, 0)))
  x0 = jnp.stack([ego[:, :DH], ego[:, DH:]], axis=0)
  x0h = lax.bitcast_convert_type(
      x0.astype(BF16).reshape(NC, npad, DH // 2, 2), I32
  )
  x0p = x0[:, :, jnp.array(PERM)]

  e = edge_index.shape[1]
  nbp = -(-e // (NS * RB * SB)) * SB
  epad = nbp * NS * RB
  src = jnp.pad(edge_index[0], (0, epad - e)).reshape(NS, nbp, RB)
  dst = jnp.pad(edge_index[1], (0, epad - e)).reshape(NS, nbp, RB)
  w = jnp.pad(edge_weight, (0, epad - e)).reshape(NS, nbp * RB)

  xsum, _, _ = _build(npad, nbp)(x0p, x0h, src, dst, w)
  xsum = xsum[:, :, jnp.array(INV_PERM)]
  mean = jnp.concatenate([xsum[0, :n], xsum[1, :n]], axis=1)
  return mean[:nu], mean[nu:]


def kernel(user_emb, item_emb, edge_index, edge_weight):
  return _lightgcn(user_emb, item_emb, edge_index, edge_weight)


# speedup vs baseline: 1.1664x; 1.1664x over previous
import functools

import jax
import jax.numpy as jnp
from jax import lax
from jax.experimental import pallas as pl
from jax.experimental.pallas import tpu as pltpu
from jax.experimental.pallas import tpu_sc as plsc

NC = 2
NS = 16
LANES = 16
DH = 32
RB = 128
SB = 8
NBUF = 5
N_LAYERS = 3
F32 = jnp.float32
I32 = jnp.int32


@functools.lru_cache(maxsize=None)
def _build(Npad, NBp):
  NSB = NBp // SB
  RPT = Npad // NS
  NRC = RPT // RB

  mesh = plsc.VectorSubcoreMesh(
      core_axis_name="c", subcore_axis_name="s", num_cores=NC, num_subcores=NS
  )

  @functools.partial(
      pl.kernel,
      out_type=(
          jax.ShapeDtypeStruct((NC, Npad, DH), F32),
          jax.ShapeDtypeStruct((NC, Npad, DH), F32),
          jax.ShapeDtypeStruct((NC, Npad, DH), F32),
      ),
      mesh=mesh,
      compiler_params=pltpu.CompilerParams(needs_layout_passes=False, use_tc_tiling_on_sc=False),
      scratch_types=dict(
          acc=pltpu.VMEM_SHARED((Npad, DH), F32),
          srcb0=pltpu.VMEM((SB, RB), I32),
          srcb1=pltpu.VMEM((SB, RB), I32),
          dstb0=pltpu.VMEM((SB, RB), I32),
          dstb1=pltpu.VMEM((SB, RB), I32),
          wb0=pltpu.VMEM((SB * RB,), F32),
          wb1=pltpu.VMEM((SB * RB,), F32),
          rows0=pltpu.VMEM((RB, DH), F32),
          rows1=pltpu.VMEM((RB, DH), F32),
          rows2=pltpu.VMEM((RB, DH), F32),
          rows3=pltpu.VMEM((RB, DH), F32),
          rows4=pltpu.VMEM((RB, DH), F32),
          gs0=pltpu.SemaphoreType.DMA,
          gs1=pltpu.SemaphoreType.DMA,
          gs2=pltpu.SemaphoreType.DMA,
          gs3=pltpu.SemaphoreType.DMA,
          gs4=pltpu.SemaphoreType.DMA,
          ss0=pltpu.SemaphoreType.DMA,
          ss1=pltpu.SemaphoreType.DMA,
          ss2=pltpu.SemaphoreType.DMA,
          ss3=pltpu.SemaphoreType.DMA,
          ss4=pltpu.SemaphoreType.DMA,
          is0=pltpu.SemaphoreType.DMA,
          is1=pltpu.SemaphoreType.DMA,
      ),
  )
  def k(x0, src3, dst3, w3, xsum, xa, xb, acc, srcb0, srcb1, dstb0, dstb1,
        wb0, wb1, rows0, rows1, rows2, rows3, rows4, gs0, gs1, gs2, gs3, gs4,
        ss0, ss1, ss2, ss3, ss4, is0, is1):
    zb = rows0
    accb = rows1
    sumb = rows2
    c = lax.axis_index("c")
    s = lax.axis_index("s")
    row0_t = s * RPT

    zeros = jnp.zeros((LANES,), F32)

    def zb_body(r, carry):
      zb[r, pl.ds(0, LANES)] = zeros
      zb[r, pl.ds(LANES, LANES)] = zeros
      return carry

    lax.fori_loop(0, RB, zb_body, 0)

    def zacc_body(kk, carry):
      pltpu.sync_copy(zb, acc.at[pl.ds(row0_t + kk * RB, RB), :])
      return carry

    lax.fori_loop(0, NRC, zacc_body, 0)
    plsc.subcore_barrier()

    splat_dnums = lax.GatherDimensionNumbers(
        offset_dims=(), collapsed_slice_dims=(0,), start_index_map=(0,)
    )
    splat_idx = [
        jnp.full((LANES, 1), j, I32) for j in range(LANES)
    ]

    def splat(vec, j):
      return lax.gather(
          vec,
          splat_idx[j],
          splat_dnums,
          (1,),
          mode=lax.GatherScatterMode.PROMISE_IN_BOUNDS,
      )

    def edge_pass(xprev):
      bufs = [rows0, rows1, rows2, rows3, rows4]
      gsems = [gs0, gs1, gs2, gs3, gs4]
      ssems = [ss0, ss1, ss2, ss3, ss4]
      srcbs = [srcb0, srcb1]
      dstbs = [dstb0, dstb1]
      wbs = [wb0, wb1]
      isems = [is0, is1]

      def idx_refs(q, slot):
        base = q * SB
        return (
            (src3.at[s, pl.ds(base, SB), :], srcbs[slot]),
            (dst3.at[s, pl.ds(base, SB), :], dstbs[slot]),
            (w3.at[s, pl.ds(base * RB, SB * RB)], wbs[slot]),
        )

      def stage(q, slot):
        for src, dst in idx_refs(q, slot):
          pltpu.async_copy(src, dst, isems[slot])

      def drain(q, slot):
        for src, dst in idx_refs(q, slot):
          pltpu.make_async_copy(src, dst, isems[slot]).wait()

      def run_block(q, slot):
        srcb = srcbs[slot]
        dstb = dstbs[slot]
        wb = wbs[slot]
        gd = {}
        sd = {}

        def start_gather(b):
          cur = b % NBUF
          gd[b] = pltpu.async_copy(
              xprev.at[c].at[srcb.at[b]], bufs[cur], gsems[cur]
          )

        for i in range(NBUF - 1):
          start_gather(i)

        for b in range(SB):
          cur = b % NBUF
          gd[b].wait()
          nb = b + NBUF - 1
          if nb < SB:
            if b >= 1:
              sd[b - 1].wait()
            start_gather(nb)

          def g_body(g, carry2, b=b, wb=wb, rbuf=bufs[cur]):
            e0 = g * LANES
            wvec = wb[pl.ds(b * RB + e0, LANES)]
            for j in range(LANES):
              w = splat(wvec, j)
              r0 = rbuf[e0 + j, pl.ds(0, LANES)]
              rbuf[e0 + j, pl.ds(0, LANES)] = r0 * w
              r1 = rbuf[e0 + j, pl.ds(LANES, LANES)]
              rbuf[e0 + j, pl.ds(LANES, LANES)] = r1 * w
            return carry2

          lax.fori_loop(0, RB // LANES, g_body, 0)
          sd[b] = pltpu.async_copy(
              bufs[cur], acc.at[dstb.at[b]], ssems[cur], add=True
          )
        for b in range(max(0, SB - NBUF + 1), SB):
          sd[b].wait()

      stage(0, 0)

      def pair_body(q2, carry):
        q = q2 * 2
        stage(q + 1, 1)
        drain(q, 0)
        run_block(q, 0)
        stage(jnp.minimum(q + 2, NSB - 1), 0)
        drain(q + 1, 1)
        run_block(q + 1, 1)
        return carry

      lax.fori_loop(0, NSB // 2, pair_body, 0)
      drain(NSB - 1, 0)

    def writeback(layer, xnew):
      final = layer == N_LAYERS - 1
      scale = jnp.float32(1.0 / (N_LAYERS + 1))
      lax.fori_loop(0, RB, zb_body, 0)

      def k_body(kk, carry):
        row0 = row0_t + kk * RB
        pltpu.sync_copy(acc.at[pl.ds(row0, RB), :], accb)
        pltpu.sync_copy(zb, acc.at[pl.ds(row0, RB), :])
        if not final:
          pltpu.sync_copy(accb, xnew.at[c, pl.ds(row0, RB), :])
        if layer == 0:
          pltpu.sync_copy(x0.at[c, pl.ds(row0, RB), :], sumb)
        else:
          pltpu.sync_copy(xsum.at[c, pl.ds(row0, RB), :], sumb)

        def r_body(r, carry2):
          t0 = sumb[r, pl.ds(0, LANES)] + accb[r, pl.ds(0, LANES)]
          t1 = sumb[r, pl.ds(LANES, LANES)] + accb[r, pl.ds(LANES, LANES)]
          if final:
            t0 = t0 * scale
            t1 = t1 * scale
          sumb[r, pl.ds(0, LANES)] = t0
          sumb[r, pl.ds(LANES, LANES)] = t1
          return carry2

        lax.fori_loop(0, RB, r_body, 0)
        pltpu.sync_copy(sumb, xsum.at[c, pl.ds(row0, RB), :])
        return carry

      lax.fori_loop(0, NRC, k_body, 0)

    xprevs = [x0, xa, xb]
    xnews = [xa, xb, xa]
    for layer in range(N_LAYERS):
      edge_pass(xprevs[layer])
      plsc.subcore_barrier()
      writeback(layer, xnews[layer])
      plsc.subcore_barrier()

  return k


@jax.jit
def _lightgcn(user_emb, item_emb, edge_index, edge_weight):
  nu = user_emb.shape[0]
  ni = item_emb.shape[0]
  n = nu + ni
  ego = jnp.concatenate([user_emb, item_emb], axis=0)
  npad = -(-n // (NS * RB)) * (NS * RB)
  ego = jnp.pad(ego, ((0, npad - n), (0, 0)))
  x0 = jnp.stack([ego[:, :DH], ego[:, DH:]], axis=0)

  e = edge_index.shape[1]
  nbp = -(-e // (NS * RB * SB * 2)) * (SB * 2)
  epad = nbp * NS * RB
  src = jnp.pad(edge_index[0], (0, epad - e)).reshape(NS, nbp, RB)
  dst = jnp.pad(edge_index[1], (0, epad - e)).reshape(NS, nbp, RB)
  w = jnp.pad(edge_weight, (0, epad - e)).reshape(NS, nbp * RB)

  xsum, _, _ = _build(npad, nbp)(x0, src, dst, w)
  mean = jnp.concatenate([xsum[0, :n], xsum[1, :n]], axis=1)
  return mean[:nu], mean[nu:]


def kernel(user_emb, item_emb, edge_index, edge_weight):
  return _lightgcn(user_emb, item_emb, edge_index, edge_weight)


# speedup vs baseline: 1.1808x; 1.0124x over previous
import functools

import jax
import jax.numpy as jnp
from jax import lax
from jax.experimental import pallas as pl
from jax.experimental.pallas import tpu as pltpu
from jax.experimental.pallas import tpu_sc as plsc

NC = 2
NS = 16
LANES = 16
DH = 32
RB = 128
SB = 8
NBUF = 5
N_LAYERS = 3
F32 = jnp.float32
I32 = jnp.int32


@functools.lru_cache(maxsize=None)
def _build(Npad, NBp):
  NSB = NBp // SB
  RPT = Npad // NS
  NRC = RPT // RB

  mesh = plsc.VectorSubcoreMesh(
      core_axis_name="c", subcore_axis_name="s", num_cores=NC, num_subcores=NS
  )

  @functools.partial(
      pl.kernel,
      out_type=(
          jax.ShapeDtypeStruct((NC, Npad, DH), F32),
          jax.ShapeDtypeStruct((NC, Npad, DH), F32),
          jax.ShapeDtypeStruct((NC, Npad, DH), F32),
      ),
      mesh=mesh,
      compiler_params=pltpu.CompilerParams(needs_layout_passes=False, use_tc_tiling_on_sc=False),
      scratch_types=dict(
          acc=pltpu.VMEM_SHARED((Npad, DH), F32),
          srcb0=pltpu.VMEM((SB, RB), I32),
          srcb1=pltpu.VMEM((SB, RB), I32),
          dstb0=pltpu.VMEM((SB, RB), I32),
          dstb1=pltpu.VMEM((SB, RB), I32),
          wb0=pltpu.VMEM((SB * RB,), F32),
          wb1=pltpu.VMEM((SB * RB,), F32),
          rows0=pltpu.VMEM((RB, DH), F32),
          rows1=pltpu.VMEM((RB, DH), F32),
          rows2=pltpu.VMEM((RB, DH), F32),
          rows3=pltpu.VMEM((RB, DH), F32),
          rows4=pltpu.VMEM((RB, DH), F32),
          gs0=pltpu.SemaphoreType.DMA,
          gs1=pltpu.SemaphoreType.DMA,
          gs2=pltpu.SemaphoreType.DMA,
          gs3=pltpu.SemaphoreType.DMA,
          gs4=pltpu.SemaphoreType.DMA,
          ss0=pltpu.SemaphoreType.DMA,
          ss1=pltpu.SemaphoreType.DMA,
          ss2=pltpu.SemaphoreType.DMA,
          ss3=pltpu.SemaphoreType.DMA,
          ss4=pltpu.SemaphoreType.DMA,
          is0=pltpu.SemaphoreType.DMA,
          is1=pltpu.SemaphoreType.DMA,
      ),
  )
  def k(x0, src3, dst3, w3, xsum, xa, xb, acc, srcb0, srcb1, dstb0, dstb1,
        wb0, wb1, rows0, rows1, rows2, rows3, rows4, gs0, gs1, gs2, gs3, gs4,
        ss0, ss1, ss2, ss3, ss4, is0, is1):
    zb = rows0
    accb = rows1
    sumb = rows2
    c = lax.axis_index("c")
    s = lax.axis_index("s")
    row0_t = s * RPT

    zeros = jnp.zeros((LANES,), F32)

    def zb_body(r, carry):
      zb[r, pl.ds(0, LANES)] = zeros
      zb[r, pl.ds(LANES, LANES)] = zeros
      return carry

    lax.fori_loop(0, RB, zb_body, 0)

    def zacc_body(kk, carry):
      pltpu.sync_copy(zb, acc.at[pl.ds(row0_t + kk * RB, RB), :])
      return carry

    lax.fori_loop(0, NRC, zacc_body, 0)
    plsc.subcore_barrier()

    splat_dnums = lax.GatherDimensionNumbers(
        offset_dims=(), collapsed_slice_dims=(0,), start_index_map=(0,)
    )
    splat_idx = [
        jnp.full((LANES, 1), j, I32) for j in range(LANES)
    ]

    def splat(vec, j):
      return lax.gather(
          vec,
          splat_idx[j],
          splat_dnums,
          (1,),
          mode=lax.GatherScatterMode.PROMISE_IN_BOUNDS,
      )

    def edge_pass(xprev):
      bufs = [rows0, rows1, rows2, rows3, rows4]
      gsems = [gs0, gs1, gs2, gs3, gs4]
      ssems = [ss0, ss1, ss2, ss3, ss4]
      srcbs = [srcb0, srcb1]
      dstbs = [dstb0, dstb1]
      wbs = [wb0, wb1]
      isems = [is0, is1]

      def idx_refs(q, slot):
        base = q * SB
        return (
            (src3.at[s, pl.ds(base, SB), :], srcbs[slot]),
            (dst3.at[s, pl.ds(base, SB), :], dstbs[slot]),
            (w3.at[s, pl.ds(base * RB, SB * RB)], wbs[slot]),
        )

      def stage(q, slot):
        for src, dst in idx_refs(q, slot):
          pltpu.async_copy(src, dst, isems[slot])

      def drain(q, slot):
        for src, dst in idx_refs(q, slot):
          pltpu.make_async_copy(src, dst, isems[slot]).wait()

      def run_block(q, slot):
        srcb = srcbs[slot]
        dstb = dstbs[slot]
        wb = wbs[slot]
        gd = {}
        sd = {}

        def start_gather(b):
          cur = b % NBUF
          gd[b] = pltpu.async_copy(
              xprev.at[c].at[srcb.at[b]], bufs[cur], gsems[cur]
          )

        for i in range(NBUF - 1):
          start_gather(i)

        for b in range(SB):
          cur = b % NBUF
          gd[b].wait()
          nb = b + NBUF - 1
          if nb < SB:
            if b >= 1:
              sd[b - 1].wait()
            start_gather(nb)

          def g_body(g, carry2, b=b, wb=wb, rbuf=bufs[cur]):
            e0 = g * LANES
            wvec = wb[pl.ds(b * RB + e0, LANES)]
            for j in range(LANES):
              w = splat(wvec, j)
              r0 = rbuf[e0 + j, pl.ds(0, LANES)]
              rbuf[e0 + j, pl.ds(0, LANES)] = r0 * w
              r1 = rbuf[e0 + j, pl.ds(LANES, LANES)]
              rbuf[e0 + j, pl.ds(LANES, LANES)] = r1 * w
            return carry2

          lax.fori_loop(0, RB // LANES, g_body, 0)
          sd[b] = pltpu.async_copy(
              bufs[cur], acc.at[dstb.at[b]], ssems[cur], add=True
          )
        for b in range(max(0, SB - NBUF + 1), SB):
          sd[b].wait()

      stage(0, 0)

      def pair_body(q2, carry):
        q = q2 * 2
        stage(q + 1, 1)
        drain(q, 0)
        run_block(q, 0)
        stage(jnp.minimum(q + 2, NSB - 1), 0)
        drain(q + 1, 1)
        run_block(q + 1, 1)
        return carry

      lax.fori_loop(0, NSB // 2, pair_body, 0)
      drain(NSB - 1, 0)

    def writeback(layer, xnew):
      final = layer == N_LAYERS - 1
      scale = jnp.float32(1.0 / (N_LAYERS + 1))
      lax.fori_loop(0, RB, zb_body, 0)

      def k_body(kk, carry):
        row0 = row0_t + kk * RB
        pltpu.sync_copy(acc.at[pl.ds(row0, RB), :], accb)
        if not final:
          wn = pltpu.async_copy(
              acc.at[pl.ds(row0, RB), :], xnew.at[c, pl.ds(row0, RB), :], ss0
          )
        if layer == 0:
          pltpu.sync_copy(x0.at[c, pl.ds(row0, RB), :], sumb)
        else:
          pltpu.sync_copy(xsum.at[c, pl.ds(row0, RB), :], sumb)

        def r_body(r, carry2):
          t0 = sumb[r, pl.ds(0, LANES)] + accb[r, pl.ds(0, LANES)]
          t1 = sumb[r, pl.ds(LANES, LANES)] + accb[r, pl.ds(LANES, LANES)]
          if final:
            t0 = t0 * scale
            t1 = t1 * scale
          sumb[r, pl.ds(0, LANES)] = t0
          sumb[r, pl.ds(LANES, LANES)] = t1
          return carry2

        lax.fori_loop(0, RB, r_body, 0)
        if not final:
          wn.wait()
        pltpu.sync_copy(sumb, xsum.at[c, pl.ds(row0, RB), :])
        return carry

      lax.fori_loop(0, NRC, k_body, 0)

      if not final:
        def zref(kk):
          return acc.at[pl.ds(row0_t + kk * RB, RB), :]

        pltpu.async_copy(zb, zref(0), ss1)

        def z_body(kk, carry):
          pltpu.async_copy(zb, zref(kk), ss1)
          pltpu.make_async_copy(zb, zref(kk - 1), ss1).wait()
          return carry

        lax.fori_loop(1, NRC, z_body, 0)
        pltpu.make_async_copy(zb, zref(NRC - 1), ss1).wait()

    xprevs = [x0, xa, xb]
    xnews = [xa, xb, xa]
    for layer in range(N_LAYERS):
      edge_pass(xprevs[layer])
      plsc.subcore_barrier()
      writeback(layer, xnews[layer])
      plsc.subcore_barrier()

  return k


@jax.jit
def _lightgcn(user_emb, item_emb, edge_index, edge_weight):
  nu = user_emb.shape[0]
  ni = item_emb.shape[0]
  n = nu + ni
  ego = jnp.concatenate([user_emb, item_emb], axis=0)
  npad = -(-n // (NS * RB)) * (NS * RB)
  ego = jnp.pad(ego, ((0, npad - n), (0, 0)))
  x0 = jnp.stack([ego[:, :DH], ego[:, DH:]], axis=0)

  e = edge_index.shape[1]
  nbp = -(-e // (NS * RB * SB * 2)) * (SB * 2)
  epad = nbp * NS * RB
  src = jnp.pad(edge_index[0], (0, epad - e)).reshape(NS, nbp, RB)
  dst = jnp.pad(edge_index[1], (0, epad - e)).reshape(NS, nbp, RB)
  w = jnp.pad(edge_weight, (0, epad - e)).reshape(NS, nbp * RB)

  xsum, _, _ = _build(npad, nbp)(x0, src, dst, w)
  mean = jnp.concatenate([xsum[0, :n], xsum[1, :n]], axis=1)
  return mean[:nu], mean[nu:]


def kernel(user_emb, item_emb, edge_index, edge_weight):
  return _lightgcn(user_emb, item_emb, edge_index, edge_weight)
